# baseline (speedup 1.0000x reference)
import functools

import jax
import jax.numpy as jnp
from jax import lax
from jax.experimental import pallas as pl
from jax.experimental.pallas import tpu as pltpu
from jax.experimental.pallas import tpu_sc as plsc

D_MODEL = 1024
B, T = 4, 2048
NW = 32
T_PER_W = T // NW
T_PER_BLK = 4
NBLK = T_PER_W // T_PER_BLK
ROWS_PER_T = (D_MODEL // 128) * B
BLK_ROWS = T_PER_BLK * ROWS_PER_T
OUT_ROWS = (T + 1) * ROWS_PER_T


def _body(ids_hbm, out_hbm, ids_all, buf0, buf1, sem0, sem1):
    nc = plsc.get_sparse_core_info().num_cores
    wid = lax.axis_index("s") * nc + lax.axis_index("c")

    pltpu.sync_copy(ids_hbm, ids_all)

    zeros16 = jnp.zeros((16,), jnp.float32)
    ones16 = jnp.ones((16,), jnp.float32)
    iota16 = lax.iota(jnp.int32, 16)
    t_l = jax.lax.shift_right_logical(iota16, 2)
    b_l = jax.lax.bitwise_and(iota16, 3)

    def _zero(bref):
        def _zinit(i, c):
            for off in range(0, 128, 16):
                bref[i, pl.ds(off, 16)] = zeros16
            return c

        lax.fori_loop(0, BLK_ROWS, _zinit, 0)

    def blk(k):
        t = wid * T_PER_W + k * T_PER_BLK + t_l
        g = plsc.load_gather(ids_all, [b_l, jnp.maximum(t - 1, 0)])
        idv = jnp.where(jnp.equal(t, 0), 0, g)
        r = t_l * ROWS_PER_T + jax.lax.shift_right_logical(idv, 7) * B + b_l
        return r, jax.lax.bitwise_and(idv, 127)

    def dst(k):
        row0 = (wid * T_PER_W + k * T_PER_BLK) * ROWS_PER_T
        return out_hbm.at[pl.ds(row0, BLK_ROWS), :]

    _zero(buf0)
    r0, c0 = blk(0)
    plsc.store_scatter(buf0, [r0, c0], ones16)
    pltpu.async_copy(buf0, dst(0), sem0)
    _zero(buf1)
    r1, c1 = blk(1)
    plsc.store_scatter(buf1, [r1, c1], ones16)
    pltpu.async_copy(buf1, dst(1), sem1)

    def _pipe(i, c):
        k0 = 2 * i
        for buf, sem, kk in ((buf0, sem0, k0), (buf1, sem1, k0 + 1)):
            pltpu.make_async_copy(buf, dst(kk - 2), sem).wait()
            ro, co = blk(kk - 2)
            plsc.store_scatter(buf, [ro, co], zeros16)
            rn, cn = blk(kk)
            plsc.store_scatter(buf, [rn, cn], ones16)
            pltpu.async_copy(buf, dst(kk), sem)
        return c

    lax.fori_loop(1, NBLK // 2, _pipe, 0)

    pltpu.make_async_copy(buf0, dst(NBLK - 2), sem0).wait()
    pltpu.make_async_copy(buf1, dst(NBLK - 1), sem1).wait()

    @pl.when(wid == 0)
    def _tail():
        old_r, old_c = blk(NBLK - 2)
        plsc.store_scatter(buf0, [old_r, old_c], zeros16)
        tid = plsc.load_gather(ids_all, [b_l, jnp.full((16,), T - 1, jnp.int32)])
        r = jax.lax.shift_right_logical(tid, 7) * B + b_l
        c = jax.lax.bitwise_and(tid, 127)
        lane = jnp.less(iota16, B)
        plsc.store_scatter(buf0, [r, c], ones16, mask=lane)
        pltpu.sync_copy(buf0.at[pl.ds(0, ROWS_PER_T)],
                        out_hbm.at[pl.ds(T * ROWS_PER_T, ROWS_PER_T), :])


@functools.partial(jax.jit, static_argnums=())
def kernel(input_ids):
    k = pl.kernel(
        _body,
        out_type=jax.ShapeDtypeStruct((OUT_ROWS, 128), jnp.float32),
        mesh=plsc.VectorSubcoreMesh(core_axis_name="c", subcore_axis_name="s"),
        compiler_params=pltpu.CompilerParams(needs_layout_passes=False),
        scratch_types=[
            pltpu.VMEM((B, T), jnp.int32),
            pltpu.VMEM((BLK_ROWS, 128), jnp.float32),
            pltpu.VMEM((BLK_ROWS, 128), jnp.float32),
            pltpu.SemaphoreType.DMA,
            pltpu.SemaphoreType.DMA,
        ],
    )
    out = k(input_ids.astype(jnp.int32))
    return (out.reshape(T + 1, D_MODEL // 128, B, 128)
               .transpose(2, 0, 1, 3)
               .reshape(B, T + 1, D_MODEL))


# speedup vs baseline: 1.0676x; 1.0676x over previous
import functools

import jax
import jax.numpy as jnp
from jax import lax
from jax.experimental import pallas as pl
from jax.experimental.pallas import tpu as pltpu
from jax.experimental.pallas import tpu_sc as plsc

D_MODEL = 1024
B, T = 4, 2048
NW = 32
T_PER_W = T // NW
T_PER_BLK = 4
NBLK = T_PER_W // T_PER_BLK
ROWS_PER_T = (D_MODEL // 128) * B
BLK_ROWS = T_PER_BLK * ROWS_PER_T
OUT_ROWS = (T + 1) * ROWS_PER_T


def _body(ids_hbm, out_hbm, ids_all, buf0, buf1, sem0, sem1):
    nc = plsc.get_sparse_core_info().num_cores
    wid = lax.axis_index("s") * nc + lax.axis_index("c")

    pltpu.sync_copy(ids_hbm, ids_all)

    zeros16 = jnp.zeros((16,), jnp.float32)
    ones16 = jnp.ones((16,), jnp.float32)
    iota16 = lax.iota(jnp.int32, 16)
    t_l = jax.lax.shift_right_logical(iota16, 2)
    b_l = jax.lax.bitwise_and(iota16, 3)

    def _zero(bref):
        def _zinit(i, c):
            for off in range(0, 128, 16):
                bref[i, pl.ds(off, 16)] = zeros16
            return c

        lax.fori_loop(0, BLK_ROWS, _zinit, 0)

    def blk(k):
        t = wid * T_PER_W + k * T_PER_BLK + t_l
        g = plsc.load_gather(ids_all, [b_l * T + jnp.maximum(t - 1, 0)])
        idv = jnp.where(jnp.equal(t, 0), 0, g)
        r = t_l * ROWS_PER_T + jax.lax.shift_right_logical(idv, 7) * B + b_l
        return r, jax.lax.bitwise_and(idv, 127)

    def dst(k):
        row0 = (wid * T_PER_W + k * T_PER_BLK) * ROWS_PER_T
        return out_hbm.at[pl.ds(row0, BLK_ROWS), :]

    _zero(buf0)
    r0, c0 = blk(0)
    plsc.store_scatter(buf0, [r0, c0], ones16)
    pltpu.async_copy(buf0, dst(0), sem0)
    _zero(buf1)
    r1, c1 = blk(1)
    plsc.store_scatter(buf1, [r1, c1], ones16)
    pltpu.async_copy(buf1, dst(1), sem1)

    def _pipe(i, c):
        k0 = 2 * i
        for buf, sem, kk in ((buf0, sem0, k0), (buf1, sem1, k0 + 1)):
            pltpu.make_async_copy(buf, dst(kk - 2), sem).wait()
            ro, co = blk(kk - 2)
            plsc.store_scatter(buf, [ro, co], zeros16)
            rn, cn = blk(kk)
            plsc.store_scatter(buf, [rn, cn], ones16)
            pltpu.async_copy(buf, dst(kk), sem)
        return c

    lax.fori_loop(1, NBLK // 2, _pipe, 0)

    pltpu.make_async_copy(buf0, dst(NBLK - 2), sem0).wait()
    pltpu.make_async_copy(buf1, dst(NBLK - 1), sem1).wait()

    @pl.when(wid == 0)
    def _tail():
        old_r, old_c = blk(NBLK - 2)
        plsc.store_scatter(buf0, [old_r, old_c], zeros16)
        tid = plsc.load_gather(ids_all, [b_l * T + (T - 1)])
        r = jax.lax.shift_right_logical(tid, 7) * B + b_l
        c = jax.lax.bitwise_and(tid, 127)
        lane = jnp.less(iota16, B)
        plsc.store_scatter(buf0, [r, c], ones16, mask=lane)
        pltpu.sync_copy(buf0.at[pl.ds(0, ROWS_PER_T)],
                        out_hbm.at[pl.ds(T * ROWS_PER_T, ROWS_PER_T), :])


@functools.partial(jax.jit, static_argnums=())
def kernel(input_ids):
    k = pl.kernel(
        _body,
        out_type=jax.ShapeDtypeStruct((OUT_ROWS, 128), jnp.float32),
        mesh=plsc.VectorSubcoreMesh(core_axis_name="c", subcore_axis_name="s"),
        compiler_params=pltpu.CompilerParams(needs_layout_passes=False),
        scratch_types=[
            pltpu.VMEM((B * T,), jnp.int32),
            pltpu.VMEM((BLK_ROWS, 128), jnp.float32),
            pltpu.VMEM((BLK_ROWS, 128), jnp.float32),
            pltpu.SemaphoreType.DMA,
            pltpu.SemaphoreType.DMA,
        ],
    )
    out = k(input_ids.astype(jnp.int32).reshape(B * T))
    return (out.reshape(T + 1, D_MODEL // 128, B, 128)
               .transpose(2, 0, 1, 3)
               .reshape(B, T + 1, D_MODEL))


# speedup vs baseline: 1.1045x; 1.0346x over previous
import functools

import jax
import jax.numpy as jnp
from jax import lax
from jax.experimental import pallas as pl
from jax.experimental.pallas import tpu as pltpu
from jax.experimental.pallas import tpu_sc as plsc

D_MODEL = 1024
B, T = 4, 2048
NW = 32
T_PER_W = T // NW
T_PER_BLK = 4
NBLK = T_PER_W // T_PER_BLK
ROWS_PER_T = (D_MODEL // 128) * B
BLK_ROWS = T_PER_BLK * ROWS_PER_T
OUT_ROWS = (T + 1) * ROWS_PER_T
IDS_PAD = (T + 1) * B + 12


def _body(ids_hbm, out_hbm, ids_v, tail_v, buf0, buf1, sem0, sem1):
    nc = plsc.get_sparse_core_info().num_cores
    wid = lax.axis_index("s") * nc + lax.axis_index("c")

    pltpu.sync_copy(ids_hbm.at[pl.ds(wid * T_PER_W * B, T_PER_W * B)], ids_v)
    pltpu.sync_copy(ids_hbm.at[pl.ds(T * B, 16)], tail_v)

    zeros16 = jnp.zeros((16,), jnp.float32)
    ones16 = jnp.ones((16,), jnp.float32)
    iota16 = lax.iota(jnp.int32, 16)
    t_l = jax.lax.shift_right_logical(iota16, 2)
    b_l = jax.lax.bitwise_and(iota16, 3)

    def _zero(bref):
        def _zinit(i, c):
            for off in range(0, 128, 16):
                bref[i, pl.ds(off, 16)] = zeros16
            return c

        lax.fori_loop(0, BLK_ROWS, _zinit, 0)

    def blk(k):
        idv = ids_v[pl.ds(k * 16, 16)]
        r = t_l * ROWS_PER_T + jax.lax.shift_right_logical(idv, 7) * B + b_l
        return r, jax.lax.bitwise_and(idv, 127)

    def dst(k):
        row0 = (wid * T_PER_W + k * T_PER_BLK) * ROWS_PER_T
        return out_hbm.at[pl.ds(row0, BLK_ROWS), :]

    _zero(buf0)
    r0, c0 = blk(0)
    plsc.store_scatter(buf0, [r0, c0], ones16)
    pltpu.async_copy(buf0, dst(0), sem0)
    _zero(buf1)
    r1, c1 = blk(1)
    plsc.store_scatter(buf1, [r1, c1], ones16)
    pltpu.async_copy(buf1, dst(1), sem1)

    def _pipe(i, c):
        k0 = 2 * i
        for buf, sem, kk in ((buf0, sem0, k0), (buf1, sem1, k0 + 1)):
            pltpu.make_async_copy(buf, dst(kk - 2), sem).wait()
            ro, co = blk(kk - 2)
            plsc.store_scatter(buf, [ro, co], zeros16)
            rn, cn = blk(kk)
            plsc.store_scatter(buf, [rn, cn], ones16)
            pltpu.async_copy(buf, dst(kk), sem)
        return c

    lax.fori_loop(1, NBLK // 2, _pipe, 0)

    pltpu.make_async_copy(buf0, dst(NBLK - 2), sem0).wait()
    pltpu.make_async_copy(buf1, dst(NBLK - 1), sem1).wait()

    @pl.when(wid == 0)
    def _tail():
        old_r, old_c = blk(NBLK - 2)
        plsc.store_scatter(buf0, [old_r, old_c], zeros16)
        tid = tail_v[pl.ds(0, 16)]
        r = jax.lax.shift_right_logical(tid, 7) * B + b_l
        c = jax.lax.bitwise_and(tid, 127)
        lane = jnp.less(iota16, B)
        plsc.store_scatter(buf0, [r, c], ones16, mask=lane)
        pltpu.sync_copy(buf0.at[pl.ds(0, ROWS_PER_T)],
                        out_hbm.at[pl.ds(T * ROWS_PER_T, ROWS_PER_T), :])


@functools.partial(jax.jit, static_argnums=())
def kernel(input_ids):
    ids32 = input_ids.astype(jnp.int32)
    padded = jnp.pad(ids32, ((0, 0), (1, 0)), constant_values=0)
    ids = jnp.pad(padded.T.reshape((T + 1) * B), (0, IDS_PAD - (T + 1) * B))
    k = pl.kernel(
        _body,
        out_type=jax.ShapeDtypeStruct((OUT_ROWS, 128), jnp.float32),
        mesh=plsc.VectorSubcoreMesh(core_axis_name="c", subcore_axis_name="s"),
        compiler_params=pltpu.CompilerParams(needs_layout_passes=False),
        scratch_types=[
            pltpu.VMEM((T_PER_W * B,), jnp.int32),
            pltpu.VMEM((16,), jnp.int32),
            pltpu.VMEM((BLK_ROWS, 128), jnp.float32),
            pltpu.VMEM((BLK_ROWS, 128), jnp.float32),
            pltpu.SemaphoreType.DMA,
            pltpu.SemaphoreType.DMA,
        ],
    )
    out = k(ids)
    return (out.reshape(T + 1, D_MODEL // 128, B, 128)
               .transpose(2, 0, 1, 3)
               .reshape(B, T + 1, D_MODEL))


# speedup vs baseline: 1.1153x; 1.0098x over previous
import functools

import jax
import jax.numpy as jnp
from jax import lax
from jax.experimental import pallas as pl
from jax.experimental.pallas import tpu as pltpu
from jax.experimental.pallas import tpu_sc as plsc

D_MODEL = 1024
B, T = 4, 2048
NW = 32
T_PER_W = T // NW
T_PER_BLK = 2
NBLK = T_PER_W // T_PER_BLK
ROWS_PER_T = (D_MODEL // 128) * B
BLK_ROWS = T_PER_BLK * ROWS_PER_T
OUT_ROWS = (T + 1) * ROWS_PER_T
IDS_PAD = (T + 1) * B + 12


def _body(ids_hbm, out_hbm, ids_v, tail_v, buf0, buf1, sem0, sem1):
    nc = plsc.get_sparse_core_info().num_cores
    wid = lax.axis_index("s") * nc + lax.axis_index("c")

    pltpu.sync_copy(ids_hbm.at[pl.ds(wid * T_PER_W * B, T_PER_W * B)],
                    ids_v.at[pl.ds(0, T_PER_W * B)])
    pltpu.sync_copy(ids_hbm.at[pl.ds(T * B, 16)], tail_v)

    zeros16 = jnp.zeros((16,), jnp.float32)
    ones16 = jnp.ones((16,), jnp.float32)
    iota16 = lax.iota(jnp.int32, 16)
    t_l = jax.lax.shift_right_logical(iota16, 2)
    b_l = jax.lax.bitwise_and(iota16, 3)

    def _zero(bref):
        def _zinit(i, c):
            for off in range(0, 128, 16):
                bref[i, pl.ds(off, 16)] = zeros16
            return c

        lax.fori_loop(0, BLK_ROWS, _zinit, 0)

    blk_mask = jnp.less(iota16, T_PER_BLK * B)

    def blk(k):
        idv = ids_v[pl.ds(k * (T_PER_BLK * B), 16)]
        r = t_l * ROWS_PER_T + jax.lax.shift_right_logical(idv, 7) * B + b_l
        return r, jax.lax.bitwise_and(idv, 127)

    def dst(k):
        row0 = (wid * T_PER_W + k * T_PER_BLK) * ROWS_PER_T
        return out_hbm.at[pl.ds(row0, BLK_ROWS), :]

    _zero(buf0)
    r0, c0 = blk(0)
    plsc.store_scatter(buf0, [r0, c0], ones16, mask=blk_mask)
    pltpu.async_copy(buf0, dst(0), sem0)
    _zero(buf1)
    r1, c1 = blk(1)
    plsc.store_scatter(buf1, [r1, c1], ones16, mask=blk_mask)
    pltpu.async_copy(buf1, dst(1), sem1)

    def _pipe(i, c):
        k0 = 2 * i
        for buf, sem, kk in ((buf0, sem0, k0), (buf1, sem1, k0 + 1)):
            pltpu.make_async_copy(buf, dst(kk - 2), sem).wait()
            ro, co = blk(kk - 2)
            plsc.store_scatter(buf, [ro, co], zeros16, mask=blk_mask)
            rn, cn = blk(kk)
            plsc.store_scatter(buf, [rn, cn], ones16, mask=blk_mask)
            pltpu.async_copy(buf, dst(kk), sem)
        return c

    lax.fori_loop(1, NBLK // 2, _pipe, 0)

    pltpu.make_async_copy(buf0, dst(NBLK - 2), sem0).wait()
    pltpu.make_async_copy(buf1, dst(NBLK - 1), sem1).wait()

    @pl.when(wid == 0)
    def _tail():
        old_r, old_c = blk(NBLK - 2)
        plsc.store_scatter(buf0, [old_r, old_c], zeros16, mask=blk_mask)
        tid = tail_v[pl.ds(0, 16)]
        r = jax.lax.shift_right_logical(tid, 7) * B + b_l
        c = jax.lax.bitwise_and(tid, 127)
        lane = jnp.less(iota16, B)
        plsc.store_scatter(buf0, [r, c], ones16, mask=lane)
        pltpu.sync_copy(buf0.at[pl.ds(0, ROWS_PER_T)],
                        out_hbm.at[pl.ds(T * ROWS_PER_T, ROWS_PER_T), :])


@functools.partial(jax.jit, static_argnums=())
def kernel(input_ids):
    ids32 = input_ids.astype(jnp.int32)
    padded = jnp.pad(ids32, ((0, 0), (1, 0)), constant_values=0)
    ids = jnp.pad(padded.T.reshape((T + 1) * B), (0, IDS_PAD - (T + 1) * B))
    k = pl.kernel(
        _body,
        out_type=jax.ShapeDtypeStruct((OUT_ROWS, 128), jnp.float32),
        mesh=plsc.VectorSubcoreMesh(core_axis_name="c", subcore_axis_name="s"),
        compiler_params=pltpu.CompilerParams(needs_layout_passes=False),
        scratch_types=[
            pltpu.VMEM((T_PER_W * B + 16,), jnp.int32),
            pltpu.VMEM((16,), jnp.int32),
            pltpu.VMEM((BLK_ROWS, 128), jnp.float32),
            pltpu.VMEM((BLK_ROWS, 128), jnp.float32),
            pltpu.SemaphoreType.DMA,
            pltpu.SemaphoreType.DMA,
        ],
    )
    out = k(ids)
    return (out.reshape(T + 1, D_MODEL // 128, B, 128)
               .transpose(2, 0, 1, 3)
               .reshape(B, T + 1, D_MODEL))
